# baseline (speedup 1.0000x reference)
import functools

import jax
import jax.numpy as jnp
from jax import lax
from jax.experimental import pallas as pl
from jax.experimental.pallas import tpu as pltpu
from jax.experimental.pallas import tpu_sc as plsc

V = 1_000_000
VP = 1_000_064
L_TOKENS = 16384
EMBED = 64
OUT = 128

NC = 2
NS = 16
NW = NC * NS
PER_W = L_TOKENS // NW
ISZ = 128
NI = PER_W // ISZ

CH = 16384
NCH = (VP + CH - 1) // CH

BLK = 40960
GRID = (V + BLK - 1) // BLK


def _sc_counts(idx):
    mesh = plsc.VectorSubcoreMesh(core_axis_name="c", subcore_axis_name="s")

    @functools.partial(
        pl.kernel,
        mesh=mesh,
        out_type=jax.ShapeDtypeStruct((NC, VP), jnp.float32),
        scratch_types=[
            pltpu.VMEM((NI, ISZ), jnp.int32),
            pltpu.VMEM((CH,), jnp.float32),
            pltpu.VMEM((ISZ,), jnp.float32),
            pltpu.VMEM_SHARED((VP,), jnp.float32),
            pltpu.SemaphoreType.DMA,
        ],
    )
    def k(idx_hbm, out_hbm, idx_v, z_v, one_v, c_sh, sem):
        cid = lax.axis_index("c")
        sid = lax.axis_index("s")
        wid = cid * NS + sid

        zero = jnp.zeros((16,), jnp.float32)
        for t in range(CH // 16):
            z_v[pl.ds(t * 16, 16)] = zero
        one = jnp.full((16,), 1.0, jnp.float32)
        for t in range(ISZ // 16):
            one_v[pl.ds(t * 16, 16)] = one

        TAILW = VP - (NCH - 1) * CH
        zc = [
            pltpu.async_copy(
                z_v, c_sh.at[pl.ds((u * NS + sid) * CH, CH)], sem
            )
            for u in range(3)
        ]

        @pl.when(sid < NCH - 1 - 3 * NS)
        def _zfull():
            pltpu.async_copy(
                z_v, c_sh.at[pl.ds((3 * NS + sid) * CH, CH)], sem
            ).wait()

        @pl.when(sid == NCH - 1 - 3 * NS)
        def _ztail():
            pltpu.async_copy(
                z_v.at[pl.ds(0, TAILW)],
                c_sh.at[pl.ds((NCH - 1) * CH, TAILW)],
                sem,
            ).wait()

        base = wid * PER_W
        for j in range(NI):
            pltpu.sync_copy(idx_hbm.at[pl.ds(base + j * ISZ, ISZ)], idx_v.at[j])
        for c in zc:
            c.wait()
        plsc.subcore_barrier()

        copies = [
            pltpu.async_copy(one_v, c_sh.at[idx_v.at[j]], sem, add=True)
            for j in range(NI)
        ]
        for cp in copies:
            cp.wait()
        plsc.subcore_barrier()

        oc = [
            pltpu.async_copy(
                c_sh.at[pl.ds((u * NS + sid) * CH, CH)],
                out_hbm.at[cid, pl.ds((u * NS + sid) * CH, CH)],
                sem,
            )
            for u in range(3)
        ]

        @pl.when(sid < NCH - 1 - 3 * NS)
        def _ofull():
            pltpu.async_copy(
                c_sh.at[pl.ds((3 * NS + sid) * CH, CH)],
                out_hbm.at[cid, pl.ds((3 * NS + sid) * CH, CH)],
                sem,
            ).wait()

        @pl.when(sid == NCH - 1 - 3 * NS)
        def _otail():
            pltpu.async_copy(
                c_sh.at[pl.ds((NCH - 1) * CH, TAILW)],
                out_hbm.at[cid, pl.ds((NCH - 1) * CH, TAILW)],
                sem,
            ).wait()

        for c in oc:
            c.wait()

    return k(idx)


def _tc_scan_tail(table_t, counts, w1, b1):
    def k(t_ref, c_ref, w_ref, b_ref, o_ref, acc_ref):
        g = pl.program_id(0)

        @pl.when(g == 0)
        def _init():
            acc_ref[...] = jnp.zeros_like(acc_ref)

        col = g * BLK + lax.broadcasted_iota(jnp.int32, (1, BLK), 1)
        valid = col < V
        c = jnp.where(valid, (c_ref[0, :] + c_ref[1, :])[None, :], 0.0)
        t = jnp.where(valid, t_ref[...], 0.0)
        acc_ref[...] += lax.dot_general(
            c, t, (((1,), (1,)), ((), ())),
            preferred_element_type=jnp.float32,
        )

        @pl.when(g == GRID - 1)
        def _tail():
            o_ref[...] = (
                lax.dot_general(
                    acc_ref[...],
                    w_ref[...],
                    (((1,), (1,)), ((), ())),
                    preferred_element_type=jnp.float32,
                )
                + b_ref[...]
            )

    return pl.pallas_call(
        k,
        grid=(GRID,),
        in_specs=[
            pl.BlockSpec((EMBED, BLK), lambda g: (0, g)),
            pl.BlockSpec((NC, BLK), lambda g: (0, g)),
            pl.BlockSpec((OUT, EMBED), lambda g: (0, 0)),
            pl.BlockSpec((1, OUT), lambda g: (0, 0)),
        ],
        out_specs=pl.BlockSpec((1, OUT), lambda g: (0, 0)),
        scratch_shapes=[pltpu.VMEM((1, EMBED), jnp.float32)],
        out_shape=jax.ShapeDtypeStruct((1, OUT), jnp.float32),
    )(table_t, counts, w1, b1)


@jax.jit
def kernel(inputs, embeddings, W1, b1):
    idx = inputs.astype(jnp.int32)
    counts = _sc_counts(idx)
    return _tc_scan_tail(embeddings.T, counts, W1, b1.reshape(1, OUT))
